# Initial kernel scaffold; baseline (speedup 1.0000x reference)
import jax
import jax.numpy as jnp
from jax.experimental import pallas as pl


def kernel(image_indices, rays, depth_mask, correction_dict):
    raise NotImplementedError("write your pallas kernel here")


# speedup vs baseline: 1.2634x; 1.2634x over previous
import functools

import jax
import jax.numpy as jnp
from jax import lax
from jax.experimental import pallas as pl
from jax.experimental.pallas import tpu as pltpu
from jax.experimental.pallas import tpu_sc as plsc

N_FRAMES = 1000
BATCH = 16384
L = 16
NC, NS = 2, 16
NW = NC * NS
BPW = BATCH // NW
GROUPS = BPW // L


def _body(idx_hbm, rays_hbm, mask_hbm, table_hbm, out_hbm,
          idx_v, mask_v, rays_v, table_v, out_v):
    wid = lax.axis_index("s") * NC + lax.axis_index("c")
    base = wid * BPW

    pltpu.sync_copy(table_hbm, table_v)
    pltpu.sync_copy(idx_hbm.at[pl.ds(base, BPW)], idx_v)
    pltpu.sync_copy(mask_hbm.at[pl.ds(base, BPW)], mask_v)
    pltpu.sync_copy(rays_hbm.at[pl.ds(base * 6, BPW * 6)], rays_v)

    lane = lax.broadcasted_iota(jnp.int32, (L,), 0)
    lane6 = lane * 6
    for g in range(GROUPS):
        sl = pl.ds(g * L, L)
        iv = idx_v[sl]
        mv = mask_v[sl]
        eff7 = jnp.where(mv == 1, iv, N_FRAMES) * 7
        rbase = lane6 + (g * L * 6)

        def tcol(c):
            return plsc.load_gather(table_v, [eff7 + c])

        def rcol(c):
            return plsc.load_gather(rays_v, [rbase + c])

        tx, ty, tz = tcol(0), tcol(1), tcol(2)
        qx, qy, qz, qw = tcol(3), tcol(4), tcol(5), tcol(6)
        ox, oy, oz = rcol(0), rcol(1), rcol(2)
        dx, dy, dz = rcol(3), rcol(4), rcol(5)

        r00 = 1.0 - 2.0 * (qy * qy + qz * qz)
        r01 = 2.0 * (qx * qy - qz * qw)
        r02 = 2.0 * (qx * qz + qy * qw)
        r10 = 2.0 * (qx * qy + qz * qw)
        r11 = 1.0 - 2.0 * (qx * qx + qz * qz)
        r12 = 2.0 * (qy * qz - qx * qw)
        r20 = 2.0 * (qx * qz - qy * qw)
        r21 = 2.0 * (qy * qz + qx * qw)
        r22 = 1.0 - 2.0 * (qx * qx + qy * qy)

        vals = (
            ox + tx,
            oy + ty,
            oz + tz,
            r00 * dx + r01 * dy + r02 * dz,
            r10 * dx + r11 * dy + r12 * dz,
            r20 * dx + r21 * dy + r22 * dz,
        )
        for c, v in enumerate(vals):
            plsc.store_scatter(out_v, [rbase + c], v)

    pltpu.sync_copy(out_v, out_hbm.at[pl.ds(base * 6, BPW * 6)])


@jax.jit
def _run(idx, rays_flat, mask, table_flat):
    mesh = plsc.VectorSubcoreMesh(core_axis_name="c", subcore_axis_name="s")
    fn = functools.partial(
        pl.kernel,
        mesh=mesh,
        out_type=jax.ShapeDtypeStruct((BATCH * 6,), jnp.float32),
        compiler_params=pltpu.CompilerParams(needs_layout_passes=False),
        scratch_types=[
            pltpu.VMEM((BPW,), jnp.int32),
            pltpu.VMEM((BPW,), jnp.int32),
            pltpu.VMEM((BPW * 6,), jnp.float32),
            pltpu.VMEM(((N_FRAMES + 1) * 7,), jnp.float32),
            pltpu.VMEM((BPW * 6,), jnp.float32),
        ],
    )(_body)
    return fn(idx, rays_flat, mask, table_flat)


def kernel(image_indices, rays, depth_mask, correction_dict):
    idx = image_indices.astype(jnp.int32)
    mask = depth_mask.reshape(BATCH).astype(jnp.int32)
    ident = jnp.array([[0.0, 0.0, 0.0, 0.0, 0.0, 0.0, 1.0]],
                      dtype=correction_dict.dtype)
    table = jnp.concatenate([correction_dict, ident], axis=0)
    out = _run(idx, rays.reshape(BATCH * 6), mask, table.reshape(-1))
    return out.reshape(BATCH, 6)


# speedup vs baseline: 1.2983x; 1.0276x over previous
import functools

import jax
import jax.numpy as jnp
from jax import lax
from jax.experimental import pallas as pl
from jax.experimental.pallas import tpu as pltpu
from jax.experimental.pallas import tpu_sc as plsc

N_FRAMES = 1000
BATCH = 16384
L = 16
NC, NS = 2, 16
NW = NC * NS
BPW = BATCH // NW
GROUPS = BPW // L


def _body(idx_hbm, rays_hbm, mask_hbm, table_hbm, out_hbm,
          idx_v, mask_v, rays_v, table_v, out_v, sem):
    wid = lax.axis_index("s") * NC + lax.axis_index("c")
    base = wid * BPW

    cps = [
        pltpu.make_async_copy(table_hbm, table_v.at[pl.ds(0, N_FRAMES * 7)], sem),
        pltpu.make_async_copy(idx_hbm.at[pl.ds(base, BPW)], idx_v, sem),
        pltpu.make_async_copy(mask_hbm.at[pl.ds(base, BPW)], mask_v, sem),
        pltpu.make_async_copy(rays_hbm.at[pl.ds(base * 6, BPW * 6)], rays_v, sem),
    ]
    for cp in cps:
        cp.start()

    lane = lax.broadcasted_iota(jnp.int32, (L,), 0)
    for cp in cps:
        cp.wait()
    plsc.store_scatter(
        table_v, [N_FRAMES * 7 + lane],
        jnp.where(lane == 6, 1.0, 0.0).astype(jnp.float32),
        mask=lane < 7,
    )
    lane6 = lane * 6
    for g in range(GROUPS):
        sl = pl.ds(g * L, L)
        iv = idx_v[sl]
        mv = mask_v[sl]
        eff7 = jnp.where(mv == 1, iv, N_FRAMES) * 7
        rbase = lane6 + (g * L * 6)

        def tcol(c):
            return plsc.load_gather(table_v, [eff7 + c])

        def rcol(c):
            return plsc.load_gather(rays_v, [rbase + c])

        tx, ty, tz = tcol(0), tcol(1), tcol(2)
        qx, qy, qz, qw = tcol(3), tcol(4), tcol(5), tcol(6)
        ox, oy, oz = rcol(0), rcol(1), rcol(2)
        dx, dy, dz = rcol(3), rcol(4), rcol(5)

        r00 = 1.0 - 2.0 * (qy * qy + qz * qz)
        r01 = 2.0 * (qx * qy - qz * qw)
        r02 = 2.0 * (qx * qz + qy * qw)
        r10 = 2.0 * (qx * qy + qz * qw)
        r11 = 1.0 - 2.0 * (qx * qx + qz * qz)
        r12 = 2.0 * (qy * qz - qx * qw)
        r20 = 2.0 * (qx * qz - qy * qw)
        r21 = 2.0 * (qy * qz + qx * qw)
        r22 = 1.0 - 2.0 * (qx * qx + qy * qy)

        vals = (
            ox + tx,
            oy + ty,
            oz + tz,
            r00 * dx + r01 * dy + r02 * dz,
            r10 * dx + r11 * dy + r12 * dz,
            r20 * dx + r21 * dy + r22 * dz,
        )
        for c, v in enumerate(vals):
            plsc.store_scatter(out_v, [rbase + c], v)

    pltpu.sync_copy(out_v, out_hbm.at[pl.ds(base * 6, BPW * 6)])


@jax.jit
def _run(idx, rays_flat, mask, table_flat):
    mesh = plsc.VectorSubcoreMesh(core_axis_name="c", subcore_axis_name="s")
    fn = functools.partial(
        pl.kernel,
        mesh=mesh,
        out_type=jax.ShapeDtypeStruct((BATCH * 6,), jnp.float32),
        compiler_params=pltpu.CompilerParams(needs_layout_passes=False),
        scratch_types=[
            pltpu.VMEM((BPW,), jnp.int32),
            pltpu.VMEM((BPW,), jnp.int32),
            pltpu.VMEM((BPW * 6,), jnp.float32),
            pltpu.VMEM((N_FRAMES * 7 + L,), jnp.float32),
            pltpu.VMEM((BPW * 6,), jnp.float32),
            pltpu.SemaphoreType.DMA,
        ],
    )(_body)
    return fn(idx, rays_flat, mask, table_flat)


def kernel(image_indices, rays, depth_mask, correction_dict):
    idx = image_indices.astype(jnp.int32)
    mask = depth_mask.reshape(BATCH).astype(jnp.int32)
    out = _run(idx, rays.reshape(BATCH * 6), mask,
               correction_dict.reshape(N_FRAMES * 7))
    return out.reshape(BATCH, 6)


# speedup vs baseline: 1.6186x; 1.2467x over previous
import functools

import jax
import jax.numpy as jnp
from jax import lax
from jax.experimental import pallas as pl
from jax.experimental.pallas import tpu as pltpu
from jax.experimental.pallas import tpu_sc as plsc

N_FRAMES = 1000
BATCH = 16384
L = 16
NC, NS = 2, 16
NW = NC * NS
BPW = BATCH // NW
HALF = BPW // 2
HGROUPS = HALF // L


def _body(idx_hbm, rays_hbm, mask_hbm, table_hbm, out_hbm,
          idx_v, mask_v, rays_v, table_v, out_v, sem):
    wid = lax.axis_index("s") * NC + lax.axis_index("c")
    base = wid * BPW

    cps = [
        pltpu.make_async_copy(table_hbm, table_v.at[pl.ds(0, N_FRAMES * 7)], sem),
        pltpu.make_async_copy(idx_hbm.at[pl.ds(base, BPW)], idx_v, sem),
        pltpu.make_async_copy(mask_hbm.at[pl.ds(base, BPW)], mask_v, sem),
        pltpu.make_async_copy(rays_hbm.at[pl.ds(base, HALF)], rays_v, sem),
    ]
    for cp in cps:
        cp.start()

    lane = lax.broadcasted_iota(jnp.int32, (L,), 0)
    for cp in cps:
        cp.wait()
    plsc.store_scatter(
        table_v, [N_FRAMES * 7 + lane],
        jnp.where(lane == 6, 1.0, 0.0).astype(jnp.float32),
        mask=lane < 7,
    )

    for h in range(2):
        for g in range(HGROUPS):
            sl = pl.ds(h * HALF + g * L, L)
            row = lane + (g * L)
            iv = idx_v[sl]
            mv = mask_v[sl]
            eff7 = jnp.where(mv == 1, iv, N_FRAMES) * 7

            def tcol(c):
                return plsc.load_gather(table_v, [eff7 + c])

            def rcol(c):
                return plsc.load_gather(rays_v, [row, jnp.full((L,), c, jnp.int32)])

            tx, ty, tz = tcol(0), tcol(1), tcol(2)
            qx, qy, qz, qw = tcol(3), tcol(4), tcol(5), tcol(6)
            ox, oy, oz = rcol(0), rcol(1), rcol(2)
            dx, dy, dz = rcol(3), rcol(4), rcol(5)

            r00 = 1.0 - 2.0 * (qy * qy + qz * qz)
            r01 = 2.0 * (qx * qy - qz * qw)
            r02 = 2.0 * (qx * qz + qy * qw)
            r10 = 2.0 * (qx * qy + qz * qw)
            r11 = 1.0 - 2.0 * (qx * qx + qz * qz)
            r12 = 2.0 * (qy * qz - qx * qw)
            r20 = 2.0 * (qx * qz - qy * qw)
            r21 = 2.0 * (qy * qz + qx * qw)
            r22 = 1.0 - 2.0 * (qx * qx + qy * qy)

            vals = (
                ox + tx,
                oy + ty,
                oz + tz,
                r00 * dx + r01 * dy + r02 * dz,
                r10 * dx + r11 * dy + r12 * dz,
                r20 * dx + r21 * dy + r22 * dz,
            )
            for c, v in enumerate(vals):
                plsc.store_scatter(out_v, [row, jnp.full((L,), c, jnp.int32)], v)

        if h == 0:
            cp = pltpu.make_async_copy(
                rays_hbm.at[pl.ds(base + HALF, HALF)], rays_v, sem)
            cp.start()
            pltpu.sync_copy(out_v, out_hbm.at[pl.ds(base, HALF)])
            cp.wait()
        else:
            pltpu.sync_copy(out_v, out_hbm.at[pl.ds(base + HALF, HALF)])


@jax.jit
def _run(idx, rays, mask, table):
    mesh = plsc.VectorSubcoreMesh(core_axis_name="c", subcore_axis_name="s")
    fn = functools.partial(
        pl.kernel,
        mesh=mesh,
        out_type=jax.ShapeDtypeStruct((BATCH, 6), jnp.float32),
        compiler_params=pltpu.CompilerParams(needs_layout_passes=False),
        scratch_types=[
            pltpu.VMEM((BPW,), jnp.int32),
            pltpu.VMEM((BPW,), jnp.int32),
            pltpu.VMEM((HALF, 6), jnp.float32),
            pltpu.VMEM((N_FRAMES * 7 + L,), jnp.float32),
            pltpu.VMEM((HALF, 6), jnp.float32),
            pltpu.SemaphoreType.DMA,
        ],
    )(_body)
    return fn(idx, rays, mask, table)


def kernel(image_indices, rays, depth_mask, correction_dict):
    idx = image_indices.astype(jnp.int32)
    mask = depth_mask.reshape(BATCH).astype(jnp.int32)
    return _run(idx, rays, mask, correction_dict.reshape(N_FRAMES * 7))


# speedup vs baseline: 2.9627x; 1.8304x over previous
import functools

import jax
import jax.numpy as jnp
from jax import lax
from jax.experimental import pallas as pl
from jax.experimental.pallas import tpu as pltpu
from jax.experimental.pallas import tpu_sc as plsc

N_FRAMES = 1000
BATCH = 16384
L = 16
NC, NS = 2, 16
NW = NC * NS
BPW = BATCH // NW
GROUPS = BPW // L


def _body(idx_hbm, rays_hbm, mask_hbm, table_hbm, out_hbm,
          idx_v, mask_v, rays_v, table_v, out_v, sem):
    wid = lax.axis_index("s") * NC + lax.axis_index("c")
    base = wid * BPW

    cps = [
        pltpu.make_async_copy(table_hbm, table_v.at[pl.ds(0, N_FRAMES * 7)], sem),
        pltpu.make_async_copy(idx_hbm.at[pl.ds(base, BPW)], idx_v, sem),
        pltpu.make_async_copy(mask_hbm.at[pl.ds(base, BPW)], mask_v, sem),
        pltpu.make_async_copy(rays_hbm.at[:, pl.ds(base, BPW)], rays_v, sem),
    ]
    for cp in cps:
        cp.start()

    lane = lax.broadcasted_iota(jnp.int32, (L,), 0)
    for cp in cps:
        cp.wait()
    plsc.store_scatter(
        table_v, [N_FRAMES * 7 + lane],
        jnp.where(lane == 6, 1.0, 0.0).astype(jnp.float32),
        mask=lane < 7,
    )

    for g in range(GROUPS):
        sl = pl.ds(g * L, L)
        iv = idx_v[sl]
        mv = mask_v[sl]
        eff7 = jnp.where(mv == 1, iv, N_FRAMES) * 7

        def tcol(c):
            return plsc.load_gather(table_v, [eff7 + c])

        tx, ty, tz = tcol(0), tcol(1), tcol(2)
        qx, qy, qz, qw = tcol(3), tcol(4), tcol(5), tcol(6)
        ox, oy, oz = rays_v[0, sl], rays_v[1, sl], rays_v[2, sl]
        dx, dy, dz = rays_v[3, sl], rays_v[4, sl], rays_v[5, sl]

        r00 = 1.0 - 2.0 * (qy * qy + qz * qz)
        r01 = 2.0 * (qx * qy - qz * qw)
        r02 = 2.0 * (qx * qz + qy * qw)
        r10 = 2.0 * (qx * qy + qz * qw)
        r11 = 1.0 - 2.0 * (qx * qx + qz * qz)
        r12 = 2.0 * (qy * qz - qx * qw)
        r20 = 2.0 * (qx * qz - qy * qw)
        r21 = 2.0 * (qy * qz + qx * qw)
        r22 = 1.0 - 2.0 * (qx * qx + qy * qy)

        out_v[0, sl] = ox + tx
        out_v[1, sl] = oy + ty
        out_v[2, sl] = oz + tz
        out_v[3, sl] = r00 * dx + r01 * dy + r02 * dz
        out_v[4, sl] = r10 * dx + r11 * dy + r12 * dz
        out_v[5, sl] = r20 * dx + r21 * dy + r22 * dz

    pltpu.sync_copy(out_v, out_hbm.at[:, pl.ds(base, BPW)])


@jax.jit
def _run(idx, rays_t, mask, table):
    mesh = plsc.VectorSubcoreMesh(core_axis_name="c", subcore_axis_name="s")
    fn = functools.partial(
        pl.kernel,
        mesh=mesh,
        out_type=jax.ShapeDtypeStruct((6, BATCH), jnp.float32),
        compiler_params=pltpu.CompilerParams(needs_layout_passes=False),
        scratch_types=[
            pltpu.VMEM((BPW,), jnp.int32),
            pltpu.VMEM((BPW,), jnp.int32),
            pltpu.VMEM((6, BPW), jnp.float32),
            pltpu.VMEM((N_FRAMES * 7 + L,), jnp.float32),
            pltpu.VMEM((6, BPW), jnp.float32),
            pltpu.SemaphoreType.DMA,
        ],
    )(_body)
    return fn(idx, rays_t, mask, table)


def kernel(image_indices, rays, depth_mask, correction_dict):
    idx = image_indices.astype(jnp.int32)
    mask = depth_mask.reshape(BATCH).astype(jnp.int32)
    out_t = _run(idx, rays.T, mask, correction_dict.reshape(N_FRAMES * 7))
    return out_t.T
